# Initial kernel scaffold; baseline (speedup 1.0000x reference)
import jax
import jax.numpy as jnp
from jax.experimental import pallas as pl


def kernel(x, W_nfc, b_nfc, W_g1, b_g1, gn1_g, gn1_b, W_g2, b_g2, gn2_g, gn2_b, W_fc1, b_fc1, W_fc2, b_fc2, edge_index, batch):
    raise NotImplementedError("write your pallas kernel here")


# speedup vs baseline: 4.8631x; 4.8631x over previous
import functools

import jax
import jax.numpy as jnp
from jax import lax
from jax.experimental import pallas as pl
from jax.experimental.pallas import tpu as pltpu
from jax.experimental.pallas import tpu_sc as plsc

N = 10000
E = 320000
NUM_G = 64
D_H = 256
D_EMB = 128
NC = 2
NS = 16
_SLOPE = 0.01
_HIGH = jax.lax.Precision.HIGHEST


def _leaky(v):
    return jnp.where(v >= 0, v, v * _SLOPE)


def _mesh():
    return plsc.VectorSubcoreMesh(core_axis_name="c", subcore_axis_name="s",
                                  num_cores=NC, num_subcores=NS)


_DEG_C = 400
_DEG_EPT = E // NC // NS
_DEG_STEPS = _DEG_EPT // _DEG_C
_NP = 10240
_DPT = _NP // NS


def _deg_body(dst_hbm, out_hbm, didx_v, ones_v, zb_v, S_sp):
    c = lax.axis_index("c")
    s = lax.axis_index("s")

    def fill(i, carry):
        ones_v[pl.ds(i * 16, 16)] = jnp.ones((16,), jnp.float32)
        return carry

    lax.fori_loop(0, _DEG_C // 16, fill, 0)

    def fillz(i, carry):
        zb_v[pl.ds(i * 16, 16)] = jnp.zeros((16,), jnp.float32)
        return carry

    lax.fori_loop(0, _DPT // 16, fillz, 0)

    pltpu.sync_copy(zb_v, S_sp.at[pl.ds(s * _DPT, _DPT)])
    plsc.subcore_barrier()

    base = (c * NS + s) * _DEG_EPT

    def step(k, carry):
        pltpu.sync_copy(dst_hbm.at[pl.ds(base + k * _DEG_C, _DEG_C)], didx_v)
        pltpu.sync_copy(ones_v, S_sp.at[didx_v], add=True)
        return carry

    lax.fori_loop(0, _DEG_STEPS, step, 0)

    plsc.subcore_barrier()
    pltpu.sync_copy(S_sp.at[pl.ds(s * _DPT, _DPT)], zb_v)
    pltpu.sync_copy(zb_v, out_hbm.at[pl.ds(c * _NP + s * _DPT, _DPT)])


@functools.cache
def _deg_build():
    return pl.kernel(
        _deg_body,
        out_type=jax.ShapeDtypeStruct((NC * _NP,), jnp.float32),
        mesh=_mesh(),
        scratch_types=[
            pltpu.VMEM((_DEG_C,), jnp.int32),
            pltpu.VMEM((_DEG_C,), jnp.float32),
            pltpu.VMEM((_DPT,), jnp.float32),
            pltpu.VMEM_SHARED((_NP,), jnp.float32),
        ],
    )


_CE = 80
_EPT = E // (NC * NS)
_NCH = _EPT // _CE
_FLEL = _CE * 128
_SUB = 512
_NSUB = _FLEL // _SUB
_FLAT = N * 128
_ZC = 16000


def _esc_body(y_hbm, src_hbm, dst_hbm, out_hbm, *rest):
    sidx_v, didx_v, rows_v, flat_v, zb_v = rest[:5]
    eidx = rest[5:5 + _NSUB]
    S_sp, gsem, ssem = rest[5 + _NSUB:]
    c = lax.axis_index("c")
    s = lax.axis_index("s")

    def fz(i, carry):
        zb_v[pl.ds(i * 16, 16)] = jnp.zeros((16,), jnp.float32)
        return carry

    lax.fori_loop(0, _ZC // 16, fz, 0)

    z0 = s * (_FLAT // NS)

    def zc(k, carry):
        pltpu.sync_copy(zb_v, S_sp.at[pl.ds(z0 + k * _ZC, _ZC)])
        return carry

    lax.fori_loop(0, (_FLAT // NS) // _ZC, zc, 0)

    plsc.subcore_barrier()

    base = (c * NS + s) * _EPT
    iot = lax.iota(jnp.int32, 16)
    cgv = [iot + cg * 16 for cg in range(8)]

    def chunk(k, carry):
        off = base + k * _CE
        pltpu.sync_copy(src_hbm.at[pl.ds(off, _CE)], sidx_v)
        pltpu.sync_copy(dst_hbm.at[pl.ds(off, _CE)], didx_v)
        pltpu.async_copy(y_hbm.at[sidx_v], rows_v, gsem).wait()
        for g in range(_CE // 16):
            dv = didx_v[pl.ds(g * 16, 16)] * 128
            for l in range(16):
                e = g * 16 + l
                b = jnp.broadcast_to(lax.slice_in_dim(dv, l, l + 1), (16,))
                for cg in range(8):
                    j = (e * 8 + cg) * 16
                    eidx[j // _SUB][pl.ds(j % _SUB, 16)] = b + cgv[cg]
                    flat_v[pl.ds(j, 16)] = rows_v[e, pl.ds(cg * 16, 16)]
        descs = [
            pltpu.async_copy(flat_v.at[pl.ds(i * _SUB, _SUB)],
                             S_sp.at[eidx[i]], ssem, add=True)
            for i in range(_NSUB)
        ]
        for d in descs:
            d.wait()
        return carry

    lax.fori_loop(0, _NCH, chunk, 0)

    plsc.subcore_barrier()

    def wb(k, carry):
        sl = pl.ds(z0 + k * _ZC, _ZC)
        pltpu.sync_copy(S_sp.at[sl], zb_v)
        pltpu.sync_copy(zb_v, out_hbm.at[pl.ds(c * _FLAT + z0 + k * _ZC, _ZC)])
        return carry

    lax.fori_loop(0, (_FLAT // NS) // _ZC, wb, 0)


@functools.cache
def _esc_build():
    return pl.kernel(
        _esc_body,
        out_type=jax.ShapeDtypeStruct((NC * _FLAT,), jnp.float32),
        mesh=_mesh(),
        scratch_types=[
            pltpu.VMEM((_CE,), jnp.int32),
            pltpu.VMEM((_CE,), jnp.int32),
            pltpu.VMEM((_CE, 128), jnp.float32),
            pltpu.VMEM((_FLEL,), jnp.float32),
            pltpu.VMEM((_ZC,), jnp.float32),
            *[pltpu.VMEM((_SUB,), jnp.int32) for _ in range(_NSUB)],
            pltpu.VMEM_SHARED((_FLAT,), jnp.float32),
            pltpu.SemaphoreType.DMA,
            pltpu.SemaphoreType.DMA,
        ],
    )


_R = 2000
_NB = N // _R


def _dot(a, b):
    return jnp.dot(a, b, precision=_HIGH, preferred_element_type=jnp.float32)


def _rows(shape):
    if len(shape) == 3:
        return pl.BlockSpec(shape, lambda i: (0, i, 0))
    return pl.BlockSpec(shape, lambda i: (i, 0))


def _full(shape):
    return pl.BlockSpec(shape, lambda i: tuple(0 for _ in shape))


def _k1_body(x_ref, wn_ref, bn_ref, wg1_ref, d0_ref, d1_ref,
             ya_ref, yb_ref, dis_ref):
    deg = d0_ref[...] + d1_ref[...] + 1.0
    dis = lax.rsqrt(deg)
    h = _leaky(_dot(x_ref[...], wn_ref[...]) + bn_ref[...])
    y = _dot(h, wg1_ref[...]) * dis
    ya_ref[...] = y[:, :128]
    yb_ref[...] = y[:, 128:]
    dis_ref[...] = dis


_k1_call = pl.pallas_call(
    _k1_body,
    grid=(_NB,),
    in_specs=[
        _rows((_R, 128)),
        _full((128, D_H)),
        _full((1, D_H)),
        _full((D_H, D_H)),
        _rows((_R, 1)),
        _rows((_R, 1)),
    ],
    out_specs=[_rows((_R, 128)), _rows((_R, 128)), _rows((_R, 1))],
    out_shape=[
        jax.ShapeDtypeStruct((N, 128), jnp.float32),
        jax.ShapeDtypeStruct((N, 128), jnp.float32),
        jax.ShapeDtypeStruct((N, 1), jnp.float32),
    ],
)


def _k3_body(Sa_ref, Sb_ref, ya_ref, yb_ref, dis_ref, bg1_ref, g1_ref,
             be1_ref, wg2_ref, y2_ref):
    dis = dis_ref[...]
    a = dis * (Sa_ref[0] + Sa_ref[1] + ya_ref[...]) + bg1_ref[:, :128]
    b = dis * (Sb_ref[0] + Sb_ref[1] + yb_ref[...]) + bg1_ref[:, 128:]
    m = (jnp.sum(a, 1, keepdims=True) + jnp.sum(b, 1, keepdims=True)) * (1.0 / D_H)
    am = a - m
    bm = b - m
    var = (jnp.sum(am * am, 1, keepdims=True)
           + jnp.sum(bm * bm, 1, keepdims=True)) * (1.0 / D_H)
    inv = lax.rsqrt(var + 1e-5)
    ha = _leaky(am * inv * g1_ref[:, :128] + be1_ref[:, :128])
    hb = _leaky(bm * inv * g1_ref[:, 128:] + be1_ref[:, 128:])
    y2_ref[...] = (_dot(ha, wg2_ref[:128, :]) + _dot(hb, wg2_ref[128:, :])) * dis


_k3_call = pl.pallas_call(
    _k3_body,
    grid=(_NB,),
    in_specs=[
        _rows((NC, _R, 128)),
        _rows((NC, _R, 128)),
        _rows((_R, 128)),
        _rows((_R, 128)),
        _rows((_R, 1)),
        _full((1, D_H)),
        _full((1, D_H)),
        _full((1, D_H)),
        _full((D_H, D_EMB)),
    ],
    out_specs=_rows((_R, D_EMB)),
    out_shape=jax.ShapeDtypeStruct((N, D_EMB), jnp.float32),
)


def _k5_body(S2_ref, y2_ref, dis_ref, bg2_ref, g2_ref, be2_ref,
             wf1_ref, bf1_ref, wf2_ref, bf2_ref, batch_ref, ha_ref, out_ref,
             acc_ref):
    dis = dis_ref[...]
    h = dis * (S2_ref[0] + S2_ref[1] + y2_ref[...]) + bg2_ref[...]
    m = jnp.sum(h, 1, keepdims=True) * (1.0 / D_EMB)
    hm = h - m
    var = jnp.sum(hm * hm, 1, keepdims=True) * (1.0 / D_EMB)
    inv = lax.rsqrt(var + 1e-5)
    h = _leaky(hm * inv * g2_ref[...] + be2_ref[...])
    nrm = jnp.maximum(jnp.sqrt(jnp.sum(h * h, 1, keepdims=True)), 1e-12)
    ha = h / nrm
    ha_ref[...] = ha
    gi = lax.broadcasted_iota(jnp.int32, (NUM_G, 1), 0)
    pt = (batch_ref[0] == gi).astype(jnp.float32)

    @pl.when(pl.program_id(0) == 0)
    def _():
        acc_ref[...] = jnp.zeros((NUM_G, D_EMB), jnp.float32)

    acc_ref[...] += _dot(pt, ha)

    @pl.when(pl.program_id(0) == _NB - 1)
    def _():
        hh = _leaky(_dot(acc_ref[...], wf1_ref[...]) + bf1_ref[...])
        out_ref[...] = jnp.sum(hh * wf2_ref[...], 1, keepdims=True) + bf2_ref[...]


_k5_call = pl.pallas_call(
    _k5_body,
    grid=(_NB,),
    in_specs=[
        _rows((NC, _R, D_EMB)),
        _rows((_R, D_EMB)),
        _rows((_R, 1)),
        _full((1, D_EMB)),
        _full((1, D_EMB)),
        _full((1, D_EMB)),
        _full((D_EMB, 64)),
        _full((1, 64)),
        _full((1, 64)),
        _full((1, 1)),
        pl.BlockSpec((1, 1, _R), lambda i: (i, 0, 0)),
    ],
    out_specs=[_rows((_R, D_EMB)), _full((NUM_G, 1))],
    out_shape=[
        jax.ShapeDtypeStruct((N, D_EMB), jnp.float32),
        jax.ShapeDtypeStruct((NUM_G, 1), jnp.float32),
    ],
    scratch_shapes=[pltpu.VMEM((NUM_G, D_EMB), jnp.float32)],
)


def kernel(x, W_nfc, b_nfc, W_g1, b_g1, gn1_g, gn1_b, W_g2, b_g2, gn2_g,
           gn2_b, W_fc1, b_fc1, W_fc2, b_fc2, edge_index, batch):
    src = edge_index[0]
    dst = edge_index[1]
    deg = _deg_build()(dst).reshape(NC, _NP)
    ya, yb, dis = _k1_call(x, W_nfc, b_nfc[None, :], W_g1,
                           deg[0, :N][:, None], deg[1, :N][:, None])
    esc = _esc_build()
    S1a = esc(ya, src, dst).reshape(NC, N, 128)
    S1b = esc(yb, src, dst).reshape(NC, N, 128)
    y2 = _k3_call(S1a, S1b, ya, yb, dis, b_g1[None, :], gn1_g[None, :],
                  gn1_b[None, :], W_g2)
    S2 = esc(y2, src, dst).reshape(NC, N, 128)
    ha, out = _k5_call(S2, y2, dis, b_g2[None, :], gn2_g[None, :],
                       gn2_b[None, :], W_fc1, b_fc1[None, :],
                       W_fc2.reshape(1, 64), b_fc2.reshape(1, 1),
                       batch.reshape(_NB, 1, _R))
    return (out, ha)


# speedup vs baseline: 4.9132x; 1.0103x over previous
import functools

import jax
import jax.numpy as jnp
from jax import lax
from jax.experimental import pallas as pl
from jax.experimental.pallas import tpu as pltpu
from jax.experimental.pallas import tpu_sc as plsc

N = 10000
E = 320000
NUM_G = 64
D_H = 256
D_EMB = 128
NC = 2
NS = 16
_SLOPE = 0.01
_HIGH = jax.lax.Precision.HIGHEST


def _leaky(v):
    return jnp.where(v >= 0, v, v * _SLOPE)


def _mesh():
    return plsc.VectorSubcoreMesh(core_axis_name="c", subcore_axis_name="s",
                                  num_cores=NC, num_subcores=NS)


_DEG_C = 400
_DEG_EPT = E // NC // NS
_DEG_STEPS = _DEG_EPT // _DEG_C
_NP = 10240
_DPT = _NP // NS


def _deg_body(dst_hbm, out_hbm, didx_v, ones_v, zb_v, S_sp):
    c = lax.axis_index("c")
    s = lax.axis_index("s")

    def fill(i, carry):
        ones_v[pl.ds(i * 16, 16)] = jnp.ones((16,), jnp.float32)
        return carry

    lax.fori_loop(0, _DEG_C // 16, fill, 0)

    def fillz(i, carry):
        zb_v[pl.ds(i * 16, 16)] = jnp.zeros((16,), jnp.float32)
        return carry

    lax.fori_loop(0, _DPT // 16, fillz, 0)

    pltpu.sync_copy(zb_v, S_sp.at[pl.ds(s * _DPT, _DPT)])
    plsc.subcore_barrier()

    base = (c * NS + s) * _DEG_EPT

    def step(k, carry):
        pltpu.sync_copy(dst_hbm.at[pl.ds(base + k * _DEG_C, _DEG_C)], didx_v)
        pltpu.sync_copy(ones_v, S_sp.at[didx_v], add=True)
        return carry

    lax.fori_loop(0, _DEG_STEPS, step, 0)

    plsc.subcore_barrier()
    pltpu.sync_copy(S_sp.at[pl.ds(s * _DPT, _DPT)], zb_v)
    pltpu.sync_copy(zb_v, out_hbm.at[pl.ds(c * _NP + s * _DPT, _DPT)])


@functools.cache
def _deg_build():
    return pl.kernel(
        _deg_body,
        out_type=jax.ShapeDtypeStruct((NC * _NP,), jnp.float32),
        mesh=_mesh(),
        scratch_types=[
            pltpu.VMEM((_DEG_C,), jnp.int32),
            pltpu.VMEM((_DEG_C,), jnp.float32),
            pltpu.VMEM((_DPT,), jnp.float32),
            pltpu.VMEM_SHARED((_NP,), jnp.float32),
        ],
    )


_CE = 80
_EPT = E // (NC * NS)
_NCH = _EPT // _CE
_FLEL = _CE * 128
_SUB = 1024
_NSUB = _FLEL // _SUB
_FLAT = N * 128
_ZC = 16000


def _esc_body(y_hbm, src_hbm, dst_hbm, out_hbm, *rest):
    sidx_v, didx_v, rows_v, flat_v, zb_v = rest[:5]
    eidx = rest[5:5 + _NSUB]
    S_sp, gsem, ssem = rest[5 + _NSUB:]
    c = lax.axis_index("c")
    s = lax.axis_index("s")

    def fz(i, carry):
        zb_v[pl.ds(i * 16, 16)] = jnp.zeros((16,), jnp.float32)
        return carry

    lax.fori_loop(0, _ZC // 16, fz, 0)

    z0 = s * (_FLAT // NS)

    def zc(k, carry):
        pltpu.sync_copy(zb_v, S_sp.at[pl.ds(z0 + k * _ZC, _ZC)])
        return carry

    lax.fori_loop(0, (_FLAT // NS) // _ZC, zc, 0)

    plsc.subcore_barrier()

    base = (c * NS + s) * _EPT
    iot = lax.iota(jnp.int32, 16)
    cgv = [iot + cg * 16 for cg in range(8)]

    def chunk(k, carry):
        off = base + k * _CE
        pltpu.sync_copy(src_hbm.at[pl.ds(off, _CE)], sidx_v)
        pltpu.sync_copy(dst_hbm.at[pl.ds(off, _CE)], didx_v)
        pltpu.async_copy(y_hbm.at[sidx_v], rows_v, gsem).wait()
        for g in range(_CE // 16):
            dv = didx_v[pl.ds(g * 16, 16)] * 128
            for l in range(16):
                e = g * 16 + l
                b = jnp.broadcast_to(lax.slice_in_dim(dv, l, l + 1), (16,))
                for cg in range(8):
                    j = (e * 8 + cg) * 16
                    eidx[j // _SUB][pl.ds(j % _SUB, 16)] = b + cgv[cg]
                    flat_v[pl.ds(j, 16)] = rows_v[e, pl.ds(cg * 16, 16)]
        descs = [
            pltpu.async_copy(flat_v.at[pl.ds(i * _SUB, _SUB)],
                             S_sp.at[eidx[i]], ssem, add=True)
            for i in range(_NSUB)
        ]
        for d in descs:
            d.wait()
        return carry

    lax.fori_loop(0, _NCH, chunk, 0)

    plsc.subcore_barrier()

    def wb(k, carry):
        sl = pl.ds(z0 + k * _ZC, _ZC)
        pltpu.sync_copy(S_sp.at[sl], zb_v)
        pltpu.sync_copy(zb_v, out_hbm.at[pl.ds(c * _FLAT + z0 + k * _ZC, _ZC)])
        return carry

    lax.fori_loop(0, (_FLAT // NS) // _ZC, wb, 0)


@functools.cache
def _esc_build():
    return pl.kernel(
        _esc_body,
        out_type=jax.ShapeDtypeStruct((NC * _FLAT,), jnp.float32),
        mesh=_mesh(),
        scratch_types=[
            pltpu.VMEM((_CE,), jnp.int32),
            pltpu.VMEM((_CE,), jnp.int32),
            pltpu.VMEM((_CE, 128), jnp.float32),
            pltpu.VMEM((_FLEL,), jnp.float32),
            pltpu.VMEM((_ZC,), jnp.float32),
            *[pltpu.VMEM((_SUB,), jnp.int32) for _ in range(_NSUB)],
            pltpu.VMEM_SHARED((_FLAT,), jnp.float32),
            pltpu.SemaphoreType.DMA,
            pltpu.SemaphoreType.DMA,
        ],
    )


_R = 2000
_NB = N // _R


def _dot(a, b):
    return jnp.dot(a, b, precision=_HIGH, preferred_element_type=jnp.float32)


def _rows(shape):
    if len(shape) == 3:
        return pl.BlockSpec(shape, lambda i: (0, i, 0))
    return pl.BlockSpec(shape, lambda i: (i, 0))


def _full(shape):
    return pl.BlockSpec(shape, lambda i: tuple(0 for _ in shape))


def _k1_body(x_ref, wn_ref, bn_ref, wg1_ref, d0_ref, d1_ref,
             ya_ref, yb_ref, dis_ref):
    deg = d0_ref[...] + d1_ref[...] + 1.0
    dis = lax.rsqrt(deg)
    h = _leaky(_dot(x_ref[...], wn_ref[...]) + bn_ref[...])
    y = _dot(h, wg1_ref[...]) * dis
    ya_ref[...] = y[:, :128]
    yb_ref[...] = y[:, 128:]
    dis_ref[...] = dis


_k1_call = pl.pallas_call(
    _k1_body,
    grid=(_NB,),
    in_specs=[
        _rows((_R, 128)),
        _full((128, D_H)),
        _full((1, D_H)),
        _full((D_H, D_H)),
        _rows((_R, 1)),
        _rows((_R, 1)),
    ],
    out_specs=[_rows((_R, 128)), _rows((_R, 128)), _rows((_R, 1))],
    out_shape=[
        jax.ShapeDtypeStruct((N, 128), jnp.float32),
        jax.ShapeDtypeStruct((N, 128), jnp.float32),
        jax.ShapeDtypeStruct((N, 1), jnp.float32),
    ],
)


def _k3_body(Sa_ref, Sb_ref, ya_ref, yb_ref, dis_ref, bg1_ref, g1_ref,
             be1_ref, wg2_ref, y2_ref):
    dis = dis_ref[...]
    a = dis * (Sa_ref[0] + Sa_ref[1] + ya_ref[...]) + bg1_ref[:, :128]
    b = dis * (Sb_ref[0] + Sb_ref[1] + yb_ref[...]) + bg1_ref[:, 128:]
    m = (jnp.sum(a, 1, keepdims=True) + jnp.sum(b, 1, keepdims=True)) * (1.0 / D_H)
    am = a - m
    bm = b - m
    var = (jnp.sum(am * am, 1, keepdims=True)
           + jnp.sum(bm * bm, 1, keepdims=True)) * (1.0 / D_H)
    inv = lax.rsqrt(var + 1e-5)
    ha = _leaky(am * inv * g1_ref[:, :128] + be1_ref[:, :128])
    hb = _leaky(bm * inv * g1_ref[:, 128:] + be1_ref[:, 128:])
    y2_ref[...] = (_dot(ha, wg2_ref[:128, :]) + _dot(hb, wg2_ref[128:, :])) * dis


_k3_call = pl.pallas_call(
    _k3_body,
    grid=(_NB,),
    in_specs=[
        _rows((NC, _R, 128)),
        _rows((NC, _R, 128)),
        _rows((_R, 128)),
        _rows((_R, 128)),
        _rows((_R, 1)),
        _full((1, D_H)),
        _full((1, D_H)),
        _full((1, D_H)),
        _full((D_H, D_EMB)),
    ],
    out_specs=_rows((_R, D_EMB)),
    out_shape=jax.ShapeDtypeStruct((N, D_EMB), jnp.float32),
)


def _k5_body(S2_ref, y2_ref, dis_ref, bg2_ref, g2_ref, be2_ref,
             wf1_ref, bf1_ref, wf2_ref, bf2_ref, batch_ref, ha_ref, out_ref,
             acc_ref):
    dis = dis_ref[...]
    h = dis * (S2_ref[0] + S2_ref[1] + y2_ref[...]) + bg2_ref[...]
    m = jnp.sum(h, 1, keepdims=True) * (1.0 / D_EMB)
    hm = h - m
    var = jnp.sum(hm * hm, 1, keepdims=True) * (1.0 / D_EMB)
    inv = lax.rsqrt(var + 1e-5)
    h = _leaky(hm * inv * g2_ref[...] + be2_ref[...])
    nrm = jnp.maximum(jnp.sqrt(jnp.sum(h * h, 1, keepdims=True)), 1e-12)
    ha = h / nrm
    ha_ref[...] = ha
    gi = lax.broadcasted_iota(jnp.int32, (NUM_G, 1), 0)
    pt = (batch_ref[0] == gi).astype(jnp.float32)

    @pl.when(pl.program_id(0) == 0)
    def _():
        acc_ref[...] = jnp.zeros((NUM_G, D_EMB), jnp.float32)

    acc_ref[...] += _dot(pt, ha)

    @pl.when(pl.program_id(0) == _NB - 1)
    def _():
        hh = _leaky(_dot(acc_ref[...], wf1_ref[...]) + bf1_ref[...])
        out_ref[...] = jnp.sum(hh * wf2_ref[...], 1, keepdims=True) + bf2_ref[...]


_k5_call = pl.pallas_call(
    _k5_body,
    grid=(_NB,),
    in_specs=[
        _rows((NC, _R, D_EMB)),
        _rows((_R, D_EMB)),
        _rows((_R, 1)),
        _full((1, D_EMB)),
        _full((1, D_EMB)),
        _full((1, D_EMB)),
        _full((D_EMB, 64)),
        _full((1, 64)),
        _full((1, 64)),
        _full((1, 1)),
        pl.BlockSpec((1, 1, _R), lambda i: (i, 0, 0)),
    ],
    out_specs=[_rows((_R, D_EMB)), _full((NUM_G, 1))],
    out_shape=[
        jax.ShapeDtypeStruct((N, D_EMB), jnp.float32),
        jax.ShapeDtypeStruct((NUM_G, 1), jnp.float32),
    ],
    scratch_shapes=[pltpu.VMEM((NUM_G, D_EMB), jnp.float32)],
)


def kernel(x, W_nfc, b_nfc, W_g1, b_g1, gn1_g, gn1_b, W_g2, b_g2, gn2_g,
           gn2_b, W_fc1, b_fc1, W_fc2, b_fc2, edge_index, batch):
    src = edge_index[0]
    dst = edge_index[1]
    deg = _deg_build()(dst).reshape(NC, _NP)
    ya, yb, dis = _k1_call(x, W_nfc, b_nfc[None, :], W_g1,
                           deg[0, :N][:, None], deg[1, :N][:, None])
    esc = _esc_build()
    S1a = esc(ya, src, dst).reshape(NC, N, 128)
    S1b = esc(yb, src, dst).reshape(NC, N, 128)
    y2 = _k3_call(S1a, S1b, ya, yb, dis, b_g1[None, :], gn1_g[None, :],
                  gn1_b[None, :], W_g2)
    S2 = esc(y2, src, dst).reshape(NC, N, 128)
    ha, out = _k5_call(S2, y2, dis, b_g2[None, :], gn2_g[None, :],
                       gn2_b[None, :], W_fc1, b_fc1[None, :],
                       W_fc2.reshape(1, 64), b_fc2.reshape(1, 1),
                       batch.reshape(_NB, 1, _R))
    return (out, ha)
